# Initial kernel scaffold; baseline (speedup 1.0000x reference)
import jax
import jax.numpy as jnp
from jax.experimental import pallas as pl


def kernel(hidden_states, router_logits, w13_weight, w2_weight):
    raise NotImplementedError("write your pallas kernel here")


# speedup vs baseline: 1.0728x; 1.0728x over previous
import jax
import jax.numpy as jnp
from jax.experimental import pallas as pl
from jax.experimental.pallas import tpu as pltpu

E = 8
K = 2
D = 1024
F = 2048
T = 2048
TB = 256


def _routing_kernel(logits_ref, cw_ref):
    l = logits_ref[...]
    iota = jax.lax.broadcasted_iota(jnp.int32, (T, E), 1)
    m1 = jnp.max(l, axis=1, keepdims=True)
    i1 = jnp.min(jnp.where(l == m1, iota, E), axis=1, keepdims=True)
    lm = jnp.where(iota == i1, -jnp.inf, l)
    m2 = jnp.max(lm, axis=1, keepdims=True)
    i2 = jnp.min(jnp.where(lm == m2, iota, E), axis=1, keepdims=True)
    e2 = jnp.exp(m2 - m1)
    w1 = 1.0 / (1.0 + e2)
    w2 = e2 / (1.0 + e2)
    cw_ref[...] = jnp.where(iota == i1, w1, 0.0) + jnp.where(iota == i2, w2, 0.0)


def _moe_dense_kernel(cw_ref, x_ref, w13_ref, w2_ref, out_ref):
    e = pl.program_id(0)
    t = pl.program_id(1)
    x = x_ref[...]
    w13 = w13_ref[0]
    gu = jax.lax.dot_general(x, w13, (((1,), (1,)), ((), ())),
                             preferred_element_type=jnp.float32)
    gate = gu[:, :F]
    up = gu[:, F:]
    h = (gate * jax.lax.logistic(gate)) * up
    y = jax.lax.dot_general(h.astype(jnp.bfloat16), w2_ref[0],
                            (((1,), (1,)), ((), ())),
                            preferred_element_type=jnp.float32)
    cw = cw_ref[...]
    eiota = jax.lax.broadcasted_iota(jnp.int32, (TB, E), 1)
    wcol = jnp.sum(jnp.where(eiota == e, cw, 0.0), axis=1, keepdims=True)
    contrib = y * wcol
    rows = pl.ds(t * TB, TB)

    @pl.when(e == 0)
    def _():
        out_ref[rows, :] = contrib

    @pl.when(e > 0)
    def _():
        out_ref[rows, :] += contrib


def kernel(hidden_states, router_logits, w13_weight, w2_weight):
    cw = pl.pallas_call(
        _routing_kernel,
        out_shape=jax.ShapeDtypeStruct((T, E), jnp.float32),
    )(router_logits.astype(jnp.float32))

    x_bf = hidden_states.astype(jnp.bfloat16)
    w13_bf = w13_weight.astype(jnp.bfloat16)
    w2_bf = w2_weight.astype(jnp.bfloat16)

    nt = T // TB
    out = pl.pallas_call(
        _moe_dense_kernel,
        grid=(E, nt),
        in_specs=[
            pl.BlockSpec((TB, E), lambda e, t: (t, 0)),
            pl.BlockSpec((TB, D), lambda e, t: (t, 0)),
            pl.BlockSpec((1, 2 * F, D), lambda e, t: (e, 0, 0)),
            pl.BlockSpec((1, D, F), lambda e, t: (e, 0, 0)),
        ],
        out_specs=pl.BlockSpec((T, D), lambda e, t: (0, 0)),
        out_shape=jax.ShapeDtypeStruct((T, D), jnp.float32),
        compiler_params=pltpu.CompilerParams(
            dimension_semantics=("arbitrary", "arbitrary"),
        ),
    )(cw, x_bf, w13_bf, w2_bf)
    return out
